# baseline (speedup 1.0000x reference)
import functools

import jax
import jax.numpy as jnp
from jax import lax
from jax.experimental import pallas as pl
from jax.experimental.pallas import tpu as pltpu
from jax.experimental.pallas import tpu_sc as plsc

N_NODES = 10000
D = 128
N_EDGES = 320000

NC, NS = 2, 16
NW = NC * NS
CHUNK = 128
CHUNKS = 79
EPT = CHUNK * CHUNKS
E_PAD = EPT * NW
ECHUNK = 88
ECHUNKS = 115
E_PAD_E = NW * ECHUNK * ECHUNKS
CHUNKS_C0 = 160
CHUNKS_C1 = 2 * ECHUNKS - CHUNKS_C0
NROWB = 4
NIDXB = 8
GAHEAD = NROWB - 1
NPAD = 10240
ROWS_PER_TILE = NPAD // NS
DUMMY = N_NODES
DEGW = 128

_mesh = plsc.VectorSubcoreMesh(core_axis_name="c", subcore_axis_name="s")


@functools.partial(
    pl.kernel,
    out_type=jax.ShapeDtypeStruct((NC, NPAD, DEGW), jnp.float32),
    mesh=_mesh,
    scratch_types=[
        pltpu.VMEM((CHUNKS, CHUNK), jnp.int32),
        pltpu.VMEM((CHUNK, DEGW), jnp.float32),
        pltpu.VMEM_SHARED((NPAD, DEGW), jnp.float32),
    ],
)
def _sc_degree(dst_hbm, ones_hbm, zeros_hbm, degp_hbm, dst_v, ones_v, dacc):
    cid = lax.axis_index("c")
    sid = lax.axis_index("s")
    wid = cid * NS + sid
    sl = pl.ds(sid * ROWS_PER_TILE, ROWS_PER_TILE)
    pltpu.sync_copy(dst_hbm.at[wid], dst_v)
    pltpu.sync_copy(ones_hbm, ones_v)
    pltpu.sync_copy(zeros_hbm.at[sl], dacc.at[sl])
    plsc.subcore_barrier()

    def body(j, carry):
        pltpu.sync_copy(ones_v, dacc.at[dst_v.at[j]], add=True)
        return carry

    lax.fori_loop(0, CHUNKS, body, 0)
    plsc.subcore_barrier()
    pltpu.sync_copy(dacc.at[sl], degp_hbm.at[cid, sl])


@functools.partial(
    pl.kernel,
    out_type=jax.ShapeDtypeStruct((NC, NPAD, D), jnp.float32),
    mesh=_mesh,
    scratch_types=[
        pltpu.VMEM((NIDXB, 2, ECHUNK), jnp.int32),
        pltpu.VMEM((NROWB, ECHUNK, D), jnp.float32),
        pltpu.VMEM_SHARED((NPAD, D), jnp.float32),
        pltpu.SemaphoreType.DMA((NIDXB,)),
        pltpu.SemaphoreType.DMA((NROWB,)),
    ],
)
def _sc_edge_scatter(y_hbm, idx_hbm, zeros_hbm, zp_hbm,
                     idx_v, rows_v, acc, isem, gsem):
    cid = lax.axis_index("c")
    sid = lax.axis_index("s")
    wid = cid * NS + sid
    nch = jnp.where(cid == 0, CHUNKS_C0, CHUNKS_C1)
    sl = pl.ds(sid * ROWS_PER_TILE, ROWS_PER_TILE)
    pltpu.sync_copy(zeros_hbm.at[sl], acc.at[sl])
    plsc.subcore_barrier()

    for b in range(GAHEAD):
        pltpu.sync_copy(idx_hbm.at[wid, b], idx_v.at[b])
        pltpu.async_copy(y_hbm.at[idx_v.at[b, 0]], rows_v.at[b], gsem.at[b])
    for b in range(GAHEAD, NIDXB):
        pltpu.async_copy(idx_hbm.at[wid, b], idx_v.at[b], isem.at[b])

    def body(j, carry):
        cb = lax.rem(j, NROWB)
        ib = lax.rem(j, NIDXB)

        pltpu.make_async_copy(y_hbm.at[idx_v.at[ib, 0]], rows_v.at[cb],
                              gsem.at[cb]).wait()
        pltpu.sync_copy(rows_v.at[cb], acc.at[idx_v.at[ib, 1]], add=True)

        @pl.when(j + NIDXB < nch)
        def _():
            pltpu.async_copy(idx_hbm.at[wid, j + NIDXB], idx_v.at[ib],
                             isem.at[ib])

        @pl.when(j + GAHEAD < nch)
        def _():
            ib2 = lax.rem(j + GAHEAD, NIDXB)
            pltpu.make_async_copy(idx_hbm.at[wid, j + GAHEAD], idx_v.at[ib2],
                                  isem.at[ib2]).wait()
            pltpu.async_copy(y_hbm.at[idx_v.at[ib2, 0]],
                             rows_v.at[lax.rem(j + GAHEAD, NROWB)],
                             gsem.at[lax.rem(j + GAHEAD, NROWB)])
        return carry

    lax.fori_loop(0, nch, body, 0)
    plsc.subcore_barrier()
    pltpu.sync_copy(acc.at[sl], zp_hbm.at[cid, sl])


BLK = 1024
GRID = NPAD // BLK


def _dinv_from_degp(degp):
    deg = degp[0, :, :1] + degp[1, :, :1] + 1.0
    return lax.rsqrt(deg)


def _tc_scale_matmul_body(x_ref, w_ref, degp_ref, y_ref):
    dinv = _dinv_from_degp(degp_ref[...])
    y_ref[...] = dinv * jnp.dot(x_ref[...], w_ref[...],
                                preferred_element_type=jnp.float32)


def _tc_mid_body(zp_ref, y_ref, degp_ref, b_ref, w_ref, y2_ref):
    dinv = _dinv_from_degp(degp_ref[...])
    pre = dinv * (zp_ref[0] + zp_ref[1] + y_ref[...]) + b_ref[...]
    h = jnp.maximum(pre, 0.0)
    y2_ref[...] = dinv * jnp.dot(h, w_ref[...],
                                 preferred_element_type=jnp.float32)


def _tc_final_body(zp_ref, y_ref, degp_ref, b_ref, out_ref):
    dinv = _dinv_from_degp(degp_ref[...])
    out_ref[...] = dinv * (zp_ref[0] + zp_ref[1] + y_ref[...]) + b_ref[...]


_row_spec = pl.BlockSpec((BLK, D), lambda i: (i, 0))
_w_spec = pl.BlockSpec((D, D), lambda i: (0, 0))
_degp_spec = pl.BlockSpec((NC, BLK, DEGW), lambda i: (0, i, 0))
_zp_spec = pl.BlockSpec((NC, BLK, D), lambda i: (0, i, 0))
_b_spec = pl.BlockSpec((1, D), lambda i: (0, 0))
_f32_out = jax.ShapeDtypeStruct((NPAD, D), jnp.float32)

_tc_scale_matmul = pl.pallas_call(
    _tc_scale_matmul_body,
    grid=(GRID,),
    in_specs=[_row_spec, _w_spec, _degp_spec],
    out_specs=_row_spec,
    out_shape=_f32_out,
)

_tc_mid = pl.pallas_call(
    _tc_mid_body,
    grid=(GRID,),
    in_specs=[_zp_spec, _row_spec, _degp_spec, _b_spec, _w_spec],
    out_specs=_row_spec,
    out_shape=_f32_out,
)

_tc_final = pl.pallas_call(
    _tc_final_body,
    grid=(GRID,),
    in_specs=[_zp_spec, _row_spec, _degp_spec, _b_spec],
    out_specs=_row_spec,
    out_shape=_f32_out,
)


def kernel(x, edge_index, W1, b1, W2, b2):
    src = edge_index[0].astype(jnp.int32)
    dst = edge_index[1].astype(jnp.int32)
    pad = E_PAD - N_EDGES
    dst_t = jnp.pad(dst, (0, pad), constant_values=DUMMY).reshape(
        NW, CHUNKS, CHUNK)

    pade = E_PAD_E - N_EDGES
    src_e = jnp.pad(src, (0, pade))
    dst_e = jnp.pad(dst, (0, pade), constant_values=DUMMY)
    n0 = NS * CHUNKS_C0 * ECHUNK
    padc = CHUNKS_C0 - CHUNKS_C1

    def _split(a, fill):
        a0 = a[:n0].reshape(NS, CHUNKS_C0, ECHUNK)
        a1 = jnp.pad(a[n0:].reshape(NS, CHUNKS_C1, ECHUNK),
                     ((0, 0), (0, padc), (0, 0)), constant_values=fill)
        return jnp.concatenate([a0, a1], axis=0)

    idx_t = jnp.stack([_split(src_e, 0), _split(dst_e, DUMMY)], axis=2)

    x_pad = jnp.pad(x, ((0, NPAD - N_NODES), (0, 0)))
    b1r = b1.reshape(1, D)
    b2r = b2.reshape(1, D)

    ones_rows = jnp.ones((CHUNK, DEGW), jnp.float32)
    zeros_rows = jnp.zeros((NPAD, D), jnp.float32)

    degp = _sc_degree(dst_t, ones_rows, zeros_rows)
    y1 = _tc_scale_matmul(x_pad, W1, degp)
    z1p = _sc_edge_scatter(y1, idx_t, zeros_rows)
    y2 = _tc_mid(z1p, y1, degp, b1r, W2)
    z2p = _sc_edge_scatter(y2, idx_t, zeros_rows)
    out = _tc_final(z2p, y2, degp, b2r)
    return out[:N_NODES]


# speedup vs baseline: 1.3400x; 1.3400x over previous
import functools

import jax
import jax.numpy as jnp
from jax import lax
from jax.experimental import pallas as pl
from jax.experimental.pallas import tpu as pltpu
from jax.experimental.pallas import tpu_sc as plsc

N_NODES = 10000
D = 128
N_EDGES = 320000

NC, NS = 2, 16
NW = NC * NS
CHUNK = 128
CHUNKS = 79
EPT = CHUNK * CHUNKS
E_PAD = EPT * NW
ECHUNK = 96
ECHUNKS = 105
E_PAD_E = NW * ECHUNK * ECHUNKS
CHUNKS_C0 = 153
CHUNKS_C1 = 2 * ECHUNKS - CHUNKS_C0
NROWB = 3
NIDXB = 6
GAHEAD = NROWB - 1
NPAD = 10240
ROWS_PER_TILE = NPAD // NS
DUMMY = N_NODES
DEGW = 128

_mesh = plsc.VectorSubcoreMesh(core_axis_name="c", subcore_axis_name="s")


@functools.partial(
    pl.kernel,
    out_type=jax.ShapeDtypeStruct((NC, NPAD, DEGW), jnp.float32),
    mesh=_mesh,
    scratch_types=[
        pltpu.VMEM((CHUNKS, CHUNK), jnp.int32),
        pltpu.VMEM((CHUNK, DEGW), jnp.float32),
        pltpu.VMEM_SHARED((NPAD, DEGW), jnp.float32),
    ],
)
def _sc_degree(dst_hbm, ones_hbm, zeros_hbm, degp_hbm, dst_v, ones_v, dacc):
    cid = lax.axis_index("c")
    sid = lax.axis_index("s")
    wid = cid * NS + sid
    sl = pl.ds(sid * ROWS_PER_TILE, ROWS_PER_TILE)
    pltpu.sync_copy(dst_hbm.at[wid], dst_v)
    pltpu.sync_copy(ones_hbm, ones_v)
    pltpu.sync_copy(zeros_hbm.at[sl], dacc.at[sl])
    plsc.subcore_barrier()

    def body(j, carry):
        pltpu.sync_copy(ones_v, dacc.at[dst_v.at[j]], add=True)
        return carry

    lax.fori_loop(0, CHUNKS, body, 0)
    plsc.subcore_barrier()
    pltpu.sync_copy(dacc.at[sl], degp_hbm.at[cid, sl])


@functools.partial(
    pl.kernel,
    out_type=jax.ShapeDtypeStruct((NC, NPAD, D), jnp.float32),
    mesh=_mesh,
    scratch_types=[
        pltpu.VMEM((NIDXB, 2, ECHUNK), jnp.int32),
        pltpu.VMEM((NROWB, ECHUNK, D), jnp.float32),
        pltpu.VMEM_SHARED((NPAD, D), jnp.float32),
        pltpu.SemaphoreType.DMA((NIDXB,)),
        pltpu.SemaphoreType.DMA((NROWB,)),
    ],
)
def _sc_edge_scatter(y_hbm, idx_hbm, zeros_hbm, zp_hbm,
                     idx_v, rows_v, acc, isem, gsem):
    cid = lax.axis_index("c")
    sid = lax.axis_index("s")
    wid = cid * NS + sid
    nch = jnp.where(cid == 0, CHUNKS_C0, CHUNKS_C1)
    sl = pl.ds(sid * ROWS_PER_TILE, ROWS_PER_TILE)
    pltpu.sync_copy(zeros_hbm.at[sl], acc.at[sl])
    plsc.subcore_barrier()

    for b in range(GAHEAD):
        pltpu.sync_copy(idx_hbm.at[wid, b], idx_v.at[b])
        pltpu.async_copy(y_hbm.at[idx_v.at[b, 0]], rows_v.at[b], gsem.at[b])
    for b in range(GAHEAD, NIDXB):
        pltpu.async_copy(idx_hbm.at[wid, b], idx_v.at[b], isem.at[b])

    def body(j, carry):
        cb = lax.rem(j, NROWB)
        ib = lax.rem(j, NIDXB)

        pltpu.make_async_copy(y_hbm.at[idx_v.at[ib, 0]], rows_v.at[cb],
                              gsem.at[cb]).wait()
        pltpu.sync_copy(rows_v.at[cb], acc.at[idx_v.at[ib, 1]], add=True)

        @pl.when(j + NIDXB < nch)
        def _():
            pltpu.async_copy(idx_hbm.at[wid, j + NIDXB], idx_v.at[ib],
                             isem.at[ib])

        @pl.when(j + GAHEAD < nch)
        def _():
            ib2 = lax.rem(j + GAHEAD, NIDXB)
            pltpu.make_async_copy(idx_hbm.at[wid, j + GAHEAD], idx_v.at[ib2],
                                  isem.at[ib2]).wait()
            pltpu.async_copy(y_hbm.at[idx_v.at[ib2, 0]],
                             rows_v.at[lax.rem(j + GAHEAD, NROWB)],
                             gsem.at[lax.rem(j + GAHEAD, NROWB)])
        return carry

    lax.fori_loop(0, nch, body, 0)
    plsc.subcore_barrier()
    pltpu.sync_copy(acc.at[sl], zp_hbm.at[cid, sl])


BLK = 1024
GRID = NPAD // BLK


def _dinv_from_degp(degp):
    deg = degp[0, :, :1] + degp[1, :, :1] + 1.0
    return lax.rsqrt(deg)


def _tc_scale_matmul_body(x_ref, w_ref, degp_ref, y_ref):
    dinv = _dinv_from_degp(degp_ref[...])
    y_ref[...] = dinv * jnp.dot(x_ref[...], w_ref[...],
                                preferred_element_type=jnp.float32)


def _tc_mid_body(zp_ref, y_ref, degp_ref, b_ref, w_ref, y2_ref):
    dinv = _dinv_from_degp(degp_ref[...])
    pre = dinv * (zp_ref[0] + zp_ref[1] + y_ref[...]) + b_ref[...]
    h = jnp.maximum(pre, 0.0)
    y2_ref[...] = dinv * jnp.dot(h, w_ref[...],
                                 preferred_element_type=jnp.float32)


def _tc_final_body(zp_ref, y_ref, degp_ref, b_ref, out_ref):
    dinv = _dinv_from_degp(degp_ref[...])
    out_ref[...] = dinv * (zp_ref[0] + zp_ref[1] + y_ref[...]) + b_ref[...]


_row_spec = pl.BlockSpec((BLK, D), lambda i: (i, 0))
_w_spec = pl.BlockSpec((D, D), lambda i: (0, 0))
_degp_spec = pl.BlockSpec((NC, BLK, DEGW), lambda i: (0, i, 0))
_zp_spec = pl.BlockSpec((NC, BLK, D), lambda i: (0, i, 0))
_b_spec = pl.BlockSpec((1, D), lambda i: (0, 0))
_f32_out = jax.ShapeDtypeStruct((NPAD, D), jnp.float32)

_tc_scale_matmul = pl.pallas_call(
    _tc_scale_matmul_body,
    grid=(GRID,),
    in_specs=[_row_spec, _w_spec, _degp_spec],
    out_specs=_row_spec,
    out_shape=_f32_out,
)

_tc_mid = pl.pallas_call(
    _tc_mid_body,
    grid=(GRID,),
    in_specs=[_zp_spec, _row_spec, _degp_spec, _b_spec, _w_spec],
    out_specs=_row_spec,
    out_shape=_f32_out,
)

_tc_final = pl.pallas_call(
    _tc_final_body,
    grid=(GRID,),
    in_specs=[_zp_spec, _row_spec, _degp_spec, _b_spec],
    out_specs=_row_spec,
    out_shape=_f32_out,
)


def kernel(x, edge_index, W1, b1, W2, b2):
    src = edge_index[0].astype(jnp.int32)
    dst = edge_index[1].astype(jnp.int32)
    pad = E_PAD - N_EDGES
    dst_t = jnp.pad(dst, (0, pad), constant_values=DUMMY).reshape(
        NW, CHUNKS, CHUNK)

    pade = E_PAD_E - N_EDGES
    src_e = jnp.pad(src, (0, pade))
    dst_e = jnp.pad(dst, (0, pade), constant_values=DUMMY)
    n0 = NS * CHUNKS_C0 * ECHUNK
    padc = CHUNKS_C0 - CHUNKS_C1

    def _split(a, fill):
        a0 = a[:n0].reshape(NS, CHUNKS_C0, ECHUNK)
        a1 = jnp.pad(a[n0:].reshape(NS, CHUNKS_C1, ECHUNK),
                     ((0, 0), (0, padc), (0, 0)), constant_values=fill)
        return jnp.concatenate([a0, a1], axis=0)

    idx_t = jnp.stack([_split(src_e, 0), _split(dst_e, DUMMY)], axis=2)

    x_pad = jnp.pad(x, ((0, NPAD - N_NODES), (0, 0)))
    b1r = b1.reshape(1, D)
    b2r = b2.reshape(1, D)

    ones_rows = jnp.ones((CHUNK, DEGW), jnp.float32)
    zeros_rows = jnp.zeros((NPAD, D), jnp.float32)

    degp = _sc_degree(dst_t, ones_rows, zeros_rows)
    y1 = _tc_scale_matmul(x_pad, W1, degp)
    z1p = _sc_edge_scatter(y1, idx_t, zeros_rows)
    y2 = _tc_mid(z1p, y1, degp, b1r, W2)
    z2p = _sc_edge_scatter(y2, idx_t, zeros_rows)
    out = _tc_final(z2p, y2, degp, b2r)
    return out[:N_NODES]
